# baseline (speedup 1.0000x reference)
import jax
import jax.numpy as jnp
from jax import lax
from jax.experimental import pallas as pl
from jax.experimental.pallas import tpu as pltpu
from jax.experimental.pallas import tpu_sc as plsc

BATCH = 16384
LATENT = 32
NC = 2
NS = 16
NW = NC * NS
BPW = BATCH // NW
CH = 128
NCH = BPW // CH
L = 16
GROUPS_PER_CH = CH // L
NGROUPS = BPW // L


def _body(inv_hbm, stk_hbm, w_inv_hbm, w_stk_hbm, out_hbm,
          idx_i, idx_s, rows_i, rows_s, out_v, sem):
    wid = lax.axis_index("s") * NC + lax.axis_index("c")
    base = wid * BPW

    for j in range(NCH):
        pltpu.sync_copy(inv_hbm.at[pl.ds(base + j * CH, CH)], idx_i.at[j])
        pltpu.sync_copy(stk_hbm.at[pl.ds(base + j * CH, CH)], idx_s.at[j])

    copies = []
    for j in range(NCH):
        copies.append(pltpu.async_copy(
            w_inv_hbm.at[idx_i.at[j]], rows_i.at[pl.ds(j * CH, CH)], sem))
        copies.append(pltpu.async_copy(
            w_stk_hbm.at[idx_s.at[j]], rows_s.at[pl.ds(j * CH, CH)], sem))
    for c in copies:
        c.wait()

    lanes = lax.iota(jnp.int32, L)

    def g_body(g, carry):
        r = g * L + lanes
        acc = jnp.zeros((L,), jnp.float32)
        for d in range(LATENT):
            dv = jnp.full((L,), d, jnp.int32)
            a = plsc.load_gather(rows_i, [r, dv])
            b = plsc.load_gather(rows_s, [r, dv])
            acc = acc + a * b
        out_v[pl.ds(pl.multiple_of(g * L, L), L)] = acc
        return carry

    lax.fori_loop(0, NGROUPS, g_body, 0)

    pltpu.sync_copy(out_v, out_hbm.at[pl.ds(base, BPW)])


@jax.jit
def kernel(investor, stock, W_investor, W_stock):
    mesh = plsc.VectorSubcoreMesh(core_axis_name="c", subcore_axis_name="s")
    return pl.kernel(
        _body,
        out_type=jax.ShapeDtypeStruct((BATCH,), jnp.float32),
        mesh=mesh,
        compiler_params=pltpu.CompilerParams(
            needs_layout_passes=False, use_tc_tiling_on_sc=False),
        scratch_types=[
            pltpu.VMEM((NCH, CH), jnp.int32),
            pltpu.VMEM((NCH, CH), jnp.int32),
            pltpu.VMEM((BPW, LATENT), jnp.float32),
            pltpu.VMEM((BPW, LATENT), jnp.float32),
            pltpu.VMEM((BPW,), jnp.float32),
            pltpu.SemaphoreType.DMA,
        ],
    )(investor, stock, W_investor, W_stock)
